# baseline (speedup 1.0000x reference)
import functools

import jax
import jax.numpy as jnp
from jax import lax
from jax.experimental import pallas as pl
from jax.experimental.pallas import tpu as pltpu
from jax.experimental.pallas import tpu_sc as plsc

N = 50000
K = 32
B = 4
F = 3
EMB = 6
ROW = 16
NW = 32
CH = 40
ECH = CH * K
NCHUNKS = N // CH
TPW = -(-NCHUNKS // NW)
assert TPW % 2 == 0


def _f_table_tc(x, emb_t, fc_weight, fc_bias):
    bn = 1920

    def body(x_ref, e_ref, w_ref, b_ref, o_ref):
        w = w_ref[...]
        s = jnp.sum(w, axis=1)
        svec = jnp.concatenate([s] * B + [jnp.zeros((ROW - B * F,), jnp.float32)])
        b_i = lax.broadcasted_iota(jnp.int32, (B, ROW), 0)
        c_i = lax.broadcasted_iota(jnp.int32, (B, ROW), 1)
        sel = jnp.where((c_i // F == b_i) & (c_i < B * F), 1.0, 0.0)
        smat = sel * svec[None, :]
        wt = jnp.concatenate([w.T] * B + [jnp.zeros((EMB, ROW - B * F),
                                                    jnp.float32)], axis=1)
        bvec = jnp.concatenate([b_ref[...]] * B +
                               [jnp.zeros((ROW - B * F,), jnp.float32)])
        fx = lax.dot_general(x_ref[...], smat, (((0,), (0,)), ((), ())),
                             preferred_element_type=jnp.float32)
        fe = lax.dot_general(e_ref[...], wt, (((0,), (0,)), ((), ())),
                             preferred_element_type=jnp.float32)
        o = fx + fe + bvec[None, :]
        o_ref[...] = jnp.where(o >= 0, o, 0.2 * o)

    return pl.pallas_call(
        body,
        grid=(pl.cdiv(N, bn),),
        in_specs=[
            pl.BlockSpec((B, bn), lambda i: (0, i)),
            pl.BlockSpec((EMB, bn), lambda i: (0, i)),
            pl.BlockSpec((F, EMB), lambda i: (0, 0)),
            pl.BlockSpec((F,), lambda i: (0,)),
        ],
        out_specs=pl.BlockSpec((bn, ROW), lambda i: (i, 0)),
        out_shape=jax.ShapeDtypeStruct((N, ROW), jnp.float32),
    )(x, emb_t, fc_weight, fc_bias)


def _make_sc_kernel():
    mesh = plsc.VectorSubcoreMesh(core_axis_name="c", subcore_axis_name="s")

    @functools.partial(
        pl.kernel,
        mesh=mesh,
        out_type=jax.ShapeDtypeStruct((N, K), jnp.float32),
        scratch_types=[
            [pltpu.VMEM((CH, K), jnp.int32)] * 2,
            [pltpu.VMEM((ECH, ROW), jnp.float32)] * 2,
            [pltpu.VMEM((CH, ROW), jnp.float32)] * 2,
            [pltpu.VMEM((CH, K), jnp.float32)] * 2,
            [pltpu.SemaphoreType.DMA] * 2,
            [pltpu.SemaphoreType.DMA] * 2,
            [pltpu.SemaphoreType.DMA] * 2,
        ],
        compiler_params=pltpu.CompilerParams(
            use_tc_tiling_on_sc=False, needs_layout_passes=False),
    )
    def sc_kernel(f_hbm, nl_hbm, out_hbm, idx_v, nb_v, self_v, outb_v,
                  ssem, gsem, osem):
        wid = lax.axis_index("s") * 2 + lax.axis_index("c")
        iota = lax.iota(jnp.int32, 16)

        def stage(t, ph):
            cid = wid + t * NW

            @pl.when(cid < NCHUNKS)
            def _():
                nbase = cid * CH
                pltpu.async_copy(nl_hbm.at[pl.ds(nbase, CH)], idx_v[ph],
                                 ssem[ph])
                pltpu.async_copy(f_hbm.at[pl.ds(nbase, CH)], self_v[ph],
                                 ssem[ph])

        def fire(t, ph):
            cid = wid + t * NW

            @pl.when(cid < NCHUNKS)
            def _():
                pltpu.make_async_copy(
                    nl_hbm.at[pl.ds(0, CH)], idx_v[ph], ssem[ph]).wait()
                pltpu.make_async_copy(
                    f_hbm.at[pl.ds(0, CH)], self_v[ph], ssem[ph]).wait()
                for i in range(CH):
                    pltpu.async_copy(
                        f_hbm.at[idx_v[ph].at[i]],
                        nb_v[ph].at[pl.ds(i * K, K)],
                        gsem[ph],
                    )

        def compute(t, ph):
            cid = wid + t * NW

            @pl.when(cid < NCHUNKS)
            def _():
                pltpu.make_async_copy(
                    f_hbm.at[pl.ds(0, ECH)], nb_v[ph], gsem[ph]).wait()

                @pl.when(t >= 2)
                def _():
                    pltpu.make_async_copy(
                        outb_v[ph], out_hbm.at[pl.ds(0, CH)], osem[ph]).wait()

                def node_body(i, _):
                    e0 = i * K
                    self_row = self_v[ph][i]
                    for g in range(K // 16):
                        rows = e0 + g * 16 + iota
                        wsum = None
                        for b in range(B):
                            acc = None
                            for j in range(F):
                                c = b * F + j
                                col = jnp.full((16,), c, jnp.int32)
                                nb_c = plsc.load_gather(nb_v[ph], [rows, col])
                                d = nb_c - self_row[c]
                                acc = d * d if acc is None else acc + d * d
                            e = jnp.exp(-acc)
                            wsum = e if wsum is None else wsum + e
                        outb_v[ph][i, pl.ds(g * 16, 16)] = wsum * 0.25
                    return 0

                lax.fori_loop(0, CH, node_body, 0, unroll=2)
                pltpu.async_copy(outb_v[ph], out_hbm.at[pl.ds(cid * CH, CH)],
                                 osem[ph])

        stage(0, 0)
        fire(0, 0)
        stage(1, 1)

        def pair_body(it, _):
            t0 = it * 2
            fire(t0 + 1, 1)
            compute(t0, 0)
            stage(t0 + 2, 0)
            compute(t0 + 1, 1)
            fire(t0 + 2, 0)
            stage(t0 + 3, 1)
            return 0

        lax.fori_loop(0, TPW // 2, pair_body, 0)

        for tt, ph in ((TPW - 2, 0), (TPW - 1, 1)):
            cid = wid + tt * NW

            @pl.when(cid < NCHUNKS)
            def _():
                pltpu.make_async_copy(
                    outb_v[ph], out_hbm.at[pl.ds(0, CH)], osem[ph]).wait()

    return sc_kernel


@jax.jit
def kernel(x, neighbor_list, node_embeddings, fc_weight, fc_bias):
    f_table = _f_table_tc(x, node_embeddings.T, fc_weight, fc_bias)

    return _make_sc_kernel()(f_table, neighbor_list)


# speedup vs baseline: 1.0207x; 1.0207x over previous
import functools

import jax
import jax.numpy as jnp
from jax import lax
from jax.experimental import pallas as pl
from jax.experimental.pallas import tpu as pltpu
from jax.experimental.pallas import tpu_sc as plsc

N = 50000
K = 32
B = 4
F = 3
EMB = 6
ROW = 16
NW = 32
CH = 40
ECH = CH * K
NCHUNKS = N // CH
TPW = -(-NCHUNKS // NW)
assert TPW % 2 == 0


def _f_table_tc(x, emb_t, fc_weight, fc_bias):
    bn = 1920

    def body(x_ref, e_ref, w_ref, b_ref, o_ref):
        w = w_ref[...]
        s = jnp.sum(w, axis=1)
        svec = jnp.concatenate([s] * B + [jnp.zeros((ROW - B * F,), jnp.float32)])
        b_i = lax.broadcasted_iota(jnp.int32, (B, ROW), 0)
        c_i = lax.broadcasted_iota(jnp.int32, (B, ROW), 1)
        sel = jnp.where((c_i // F == b_i) & (c_i < B * F), 1.0, 0.0)
        smat = sel * svec[None, :]
        wt = jnp.concatenate([w.T] * B + [jnp.zeros((EMB, ROW - B * F),
                                                    jnp.float32)], axis=1)
        bvec = jnp.concatenate([b_ref[...]] * B +
                               [jnp.zeros((ROW - B * F,), jnp.float32)])
        fx = lax.dot_general(x_ref[...], smat, (((0,), (0,)), ((), ())),
                             preferred_element_type=jnp.float32)
        fe = lax.dot_general(e_ref[...], wt, (((0,), (0,)), ((), ())),
                             preferred_element_type=jnp.float32)
        o = fx + fe + bvec[None, :]
        o_ref[...] = jnp.where(o >= 0, o, 0.2 * o)

    return pl.pallas_call(
        body,
        grid=(pl.cdiv(N, bn),),
        in_specs=[
            pl.BlockSpec((B, bn), lambda i: (0, i)),
            pl.BlockSpec((EMB, bn), lambda i: (0, i)),
            pl.BlockSpec((F, EMB), lambda i: (0, 0)),
            pl.BlockSpec((F,), lambda i: (0,)),
        ],
        out_specs=pl.BlockSpec((bn, ROW), lambda i: (i, 0)),
        out_shape=jax.ShapeDtypeStruct((N, ROW), jnp.float32),
    )(x, emb_t, fc_weight, fc_bias)


def _make_sc_kernel():
    mesh = plsc.VectorSubcoreMesh(core_axis_name="c", subcore_axis_name="s")

    @functools.partial(
        pl.kernel,
        mesh=mesh,
        out_type=jax.ShapeDtypeStruct((N, K), jnp.float32),
        scratch_types=[
            [pltpu.VMEM((CH, K), jnp.int32)] * 2,
            [pltpu.VMEM((ECH, ROW), jnp.float32)] * 2,
            [pltpu.VMEM((CH, ROW), jnp.float32)] * 2,
            [pltpu.VMEM((CH, K), jnp.float32)] * 2,
            [pltpu.SemaphoreType.DMA] * 2,
            [pltpu.SemaphoreType.DMA] * 2,
            [pltpu.SemaphoreType.DMA] * 2,
        ],
        compiler_params=pltpu.CompilerParams(
            use_tc_tiling_on_sc=False, needs_layout_passes=False),
    )
    def sc_kernel(f_hbm, nl_hbm, out_hbm, idx_v, nb_v, self_v, outb_v,
                  ssem, gsem, osem):
        wid = lax.axis_index("s") * 2 + lax.axis_index("c")
        iota = lax.iota(jnp.int32, 16)

        def stage(t, ph):
            cid = wid + t * NW

            @pl.when(cid < NCHUNKS)
            def _():
                nbase = cid * CH
                pltpu.async_copy(nl_hbm.at[pl.ds(nbase, CH)], idx_v[ph],
                                 ssem[ph])
                pltpu.async_copy(f_hbm.at[pl.ds(nbase, CH)], self_v[ph],
                                 ssem[ph])

        def fire(t, ph):
            cid = wid + t * NW

            @pl.when(cid < NCHUNKS)
            def _():
                pltpu.make_async_copy(
                    nl_hbm.at[pl.ds(0, CH)], idx_v[ph], ssem[ph]).wait()
                pltpu.make_async_copy(
                    f_hbm.at[pl.ds(0, CH)], self_v[ph], ssem[ph]).wait()
                for i in range(CH):
                    pltpu.async_copy(
                        f_hbm.at[idx_v[ph].at[i]],
                        nb_v[ph].at[pl.ds(i * K, K)],
                        gsem[ph],
                    )

        def compute(t, ph):
            cid = wid + t * NW

            @pl.when(cid < NCHUNKS)
            def _():
                pltpu.make_async_copy(
                    f_hbm.at[pl.ds(0, ECH)], nb_v[ph], gsem[ph]).wait()

                @pl.when(t >= 2)
                def _():
                    pltpu.make_async_copy(
                        outb_v[ph], out_hbm.at[pl.ds(0, CH)], osem[ph]).wait()

                def node_body(i, _):
                    e0 = i * K
                    self_row = self_v[ph][i]
                    for g in range(K // 16):
                        grp = nb_v[ph].at[pl.ds(e0 + g * 16, 16), :]
                        wsum = None
                        for b in range(B):
                            acc = None
                            for j in range(F):
                                c = b * F + j
                                col = jnp.full((16,), c, jnp.int32)
                                nb_c = plsc.load_gather(grp, [iota, col])
                                d = nb_c - self_row[c]
                                acc = d * d if acc is None else acc + d * d
                            e = jnp.exp(-acc)
                            wsum = e if wsum is None else wsum + e
                        outb_v[ph][i, pl.ds(g * 16, 16)] = wsum * 0.25
                    return 0

                lax.fori_loop(0, CH, node_body, 0)
                pltpu.async_copy(outb_v[ph], out_hbm.at[pl.ds(cid * CH, CH)],
                                 osem[ph])

        stage(0, 0)
        fire(0, 0)
        stage(1, 1)

        def pair_body(it, _):
            t0 = it * 2
            fire(t0 + 1, 1)
            compute(t0, 0)
            stage(t0 + 2, 0)
            compute(t0 + 1, 1)
            fire(t0 + 2, 0)
            stage(t0 + 3, 1)
            return 0

        lax.fori_loop(0, TPW // 2, pair_body, 0)

        for tt, ph in ((TPW - 2, 0), (TPW - 1, 1)):
            cid = wid + tt * NW

            @pl.when(cid < NCHUNKS)
            def _():
                pltpu.make_async_copy(
                    outb_v[ph], out_hbm.at[pl.ds(0, CH)], osem[ph]).wait()

    return sc_kernel


@jax.jit
def kernel(x, neighbor_list, node_embeddings, fc_weight, fc_bias):
    f_table = _f_table_tc(x, node_embeddings.T, fc_weight, fc_bias)

    return _make_sc_kernel()(f_table, neighbor_list)
